# baseline (speedup 1.0000x reference)
import functools

import jax
import jax.numpy as jnp
import numpy as np
from jax import lax
from jax.experimental import pallas as pl
from jax.experimental.pallas import tpu as pltpu
from jax.experimental.pallas import tpu_sc as plsc

_N_OUT = 1024
_N_SAMPLE = [16, 32, 128]
_RADIUS = [0.1, 0.2, 0.4]

_NC, _NS, _L = 2, 16, 16
_NW = _NC * _NS


def _fps(xyz, K):
    B, N, _ = xyz.shape
    idx_out = jnp.zeros((B, K), dtype=jnp.int32)
    dists = jnp.full((B, N), 1e10, dtype=jnp.float32)
    farthest = jnp.zeros((B,), dtype=jnp.int32)

    def body(i, carry):
        idx_out, dists, farthest = carry
        idx_out = idx_out.at[:, i].set(farthest)
        cen = jnp.take_along_axis(xyz, farthest[:, None, None], axis=1)
        d = jnp.sum((xyz - cen) ** 2, axis=-1)
        dists = jnp.minimum(dists, d)
        farthest = jnp.argmax(dists, axis=-1).astype(jnp.int32)
        return (idx_out, dists, farthest)

    idx_out, _, _ = lax.fori_loop(0, K, body, (idx_out, dists, farthest))
    return idx_out


_K1, _K2, _K3 = _N_SAMPLE
_R1SQ = np.float32(_RADIUS[0] * _RADIUS[0])
_R2SQ = np.float32(_RADIUS[1] * _RADIUS[1])
_R3SQ = np.float32(_RADIUS[2] * _RADIUS[2])


def _bq_body(cx_hbm, cy_hbm, cz_hbm, xp_hbm, yp_hbm, zp_hbm, x8_hbm,
             g1_hbm, g2_hbm, g3_hbm, cnt1_hbm, cnt2_hbm, cnt3_hbm,
             xv, yv, zv, x8v, cenx, ceny, cenz, o1, o2, o3,
             s1, s2, s3, c1s, c2s, c3s):
    B, N = xp_hbm.shape
    rows_per_w = (B * _N_OUT) // _NW
    groups = rows_per_w // _L
    wid = lax.axis_index("s") * _NC + lax.axis_index("c")
    b = wid // (_NW // B)
    row0 = wid * rows_per_w

    pltpu.sync_copy(xp_hbm.at[b], xv)
    pltpu.sync_copy(yp_hbm.at[b], yv)
    pltpu.sync_copy(zp_hbm.at[b], zv)
    pltpu.sync_copy(x8_hbm.at[pl.ds(b * N * 8, N * 8)], x8v)
    pltpu.sync_copy(cx_hbm.at[pl.ds(row0, rows_per_w)], cenx)
    pltpu.sync_copy(cy_hbm.at[pl.ds(row0, rows_per_w)], ceny)
    pltpu.sync_copy(cz_hbm.at[pl.ds(row0, rows_per_w)], cenz)

    zi = jnp.zeros((_L,), jnp.int32)
    for kk in range(_K1):
        o1[pl.ds(kk * _L, _L)] = zi
    for kk in range(_K2):
        o2[pl.ds(kk * _L, _L)] = zi
    for kk in range(_K3):
        o3[pl.ds(kk * _L, _L)] = zi

    lane = lax.iota(jnp.int32, _L)
    zeros16 = jnp.zeros((_L,), jnp.int32)

    def group_body(g, _):
        cxv = cenx[pl.ds(g * _L, _L)]
        cyv = ceny[pl.ds(g * _L, _L)]
        czv = cenz[pl.ds(g * _L, _L)]

        def point_body(n, carry):
            c1, c2, c3 = carry
            nv = jnp.full((_L,), n, dtype=jnp.int32)
            dx = cxv - plsc.load_gather(xv, [nv])
            dy = cyv - plsc.load_gather(yv, [nv])
            dz = czv - plsc.load_gather(zv, [nv])
            d2 = dx * dx + dy * dy + dz * dz
            sel1 = (d2 <= _R1SQ) & (c1 < _K1)
            sel2 = (d2 <= _R2SQ) & (c2 < _K2)
            sel3 = (d2 <= _R3SQ) & (c3 < _K3)
            plsc.store_scatter(
                o1, [lane * _K1 + jnp.minimum(c1, _K1 - 1)], nv, mask=sel1)
            plsc.store_scatter(
                o2, [lane * _K2 + jnp.minimum(c2, _K2 - 1)], nv, mask=sel2)
            plsc.store_scatter(
                o3, [lane * _K3 + jnp.minimum(c3, _K3 - 1)], nv, mask=sel3)
            return (c1 + sel1.astype(jnp.int32),
                    c2 + sel2.astype(jnp.int32),
                    c3 + sel3.astype(jnp.int32))

        z = jnp.zeros((_L,), jnp.int32)
        c1, c2, c3 = lax.fori_loop(0, N, point_body, (z, z, z))

        plsc.store_scatter(o1, [lane * _K1], zeros16, mask=(c1 == 0))
        plsc.store_scatter(o2, [lane * _K2], zeros16, mask=(c2 == 0))
        plsc.store_scatter(o3, [lane * _K3], zeros16, mask=(c3 == 0))

        def fill_row(r, _):
            for o, st, K in ((o1, s1, _K1), (o2, s2, _K2), (o3, s3, _K3)):
                for c in range(K // _L):
                    slots = o[pl.ds(r * K + c * _L, _L)]
                    a8 = slots * 8
                    wbase = (r * K + c * _L) * 8
                    for f in range(8):
                        vals = plsc.load_gather(x8v, [a8 + f])
                        plsc.store_scatter(st, [wbase + lane * 8 + f], vals)
            return 0

        lax.fori_loop(0, _L, fill_row, 0)

        c1s[pl.ds(0, _L)] = c1
        c2s[pl.ds(0, _L)] = c2
        c3s[pl.ds(0, _L)] = c3
        rbase = row0 + g * _L
        pltpu.sync_copy(s1, g1_hbm.at[pl.ds(rbase * _K1 * 8, _L * _K1 * 8)])
        pltpu.sync_copy(s2, g2_hbm.at[pl.ds(rbase * _K2 * 8, _L * _K2 * 8)])
        pltpu.sync_copy(s3, g3_hbm.at[pl.ds(rbase * _K3 * 8, _L * _K3 * 8)])
        pltpu.sync_copy(c1s, cnt1_hbm.at[pl.ds(rbase, _L)])
        pltpu.sync_copy(c2s, cnt2_hbm.at[pl.ds(rbase, _L)])
        pltpu.sync_copy(c3s, cnt3_hbm.at[pl.ds(rbase, _L)])
        return 0

    lax.fori_loop(0, groups, group_body, 0)


def _ball_query_sc(centroid, xyzT, x8flat):
    B, S, _ = centroid.shape
    N = xyzT.shape[2]
    BS = B * S
    cflat = centroid.reshape(BS, 3)
    mesh = plsc.VectorSubcoreMesh(core_axis_name="c", subcore_axis_name="s")
    out = pl.kernel(
        _bq_body,
        out_type=[
            jax.ShapeDtypeStruct((BS * _K1 * 8,), jnp.float32),
            jax.ShapeDtypeStruct((BS * _K2 * 8,), jnp.float32),
            jax.ShapeDtypeStruct((BS * _K3 * 8,), jnp.float32),
            jax.ShapeDtypeStruct((BS,), jnp.int32),
            jax.ShapeDtypeStruct((BS,), jnp.int32),
            jax.ShapeDtypeStruct((BS,), jnp.int32),
        ],
        mesh=mesh,
        compiler_params=pltpu.CompilerParams(needs_layout_passes=False),
        scratch_types=[
            pltpu.VMEM((N,), jnp.float32),
            pltpu.VMEM((N,), jnp.float32),
            pltpu.VMEM((N,), jnp.float32),
            pltpu.VMEM((N * 8,), jnp.float32),
            pltpu.VMEM((BS // _NW,), jnp.float32),
            pltpu.VMEM((BS // _NW,), jnp.float32),
            pltpu.VMEM((BS // _NW,), jnp.float32),
            pltpu.VMEM((_L * _K1,), jnp.int32),
            pltpu.VMEM((_L * _K2,), jnp.int32),
            pltpu.VMEM((_L * _K3,), jnp.int32),
            pltpu.VMEM((_L * _K1 * 8,), jnp.float32),
            pltpu.VMEM((_L * _K2 * 8,), jnp.float32),
            pltpu.VMEM((_L * _K3 * 8,), jnp.float32),
            pltpu.VMEM((_L,), jnp.int32),
            pltpu.VMEM((_L,), jnp.int32),
            pltpu.VMEM((_L,), jnp.int32),
        ],
    )(cflat[:, 0], cflat[:, 1], cflat[:, 2],
      xyzT[:, 0], xyzT[:, 1], xyzT[:, 2], x8flat)
    g1, g2, g3, cnt1, cnt2, cnt3 = out
    return (g1, g2, g3,
            cnt1.reshape(B, S), cnt2.reshape(B, S), cnt3.reshape(B, S))


def _mlp_pool_body(g_ref, c_ref, flag_ref, w1_ref, b1_ref, w2_ref, b2_ref,
                   w3_ref, b3_ref, out_ref, *, S_t, K):
    g = g_ref[0].reshape(S_t * K, 8)
    w1 = w1_ref[...]
    cc = jax.lax.dot_general(c_ref[0], w1, (((1,), (0,)), ((), ())),
                             preferred_element_type=jnp.float32)
    h = jax.lax.dot_general(g, w1, (((1,), (0,)), ((), ())),
                            preferred_element_type=jnp.float32)
    bias1 = b1_ref[...][None, :] - cc
    h1 = h.reshape(S_t, K, -1) + bias1[:, None, :]
    h1 = jnp.maximum(h1.reshape(S_t * K, -1), 0.0)
    h2 = jax.lax.dot_general(h1, w2_ref[...], (((1,), (0,)), ((), ())),
                             preferred_element_type=jnp.float32)
    h2 = jnp.maximum(h2 + b2_ref[...][None, :], 0.0)
    h3 = jax.lax.dot_general(h2, w3_ref[...], (((1,), (0,)), ((), ())),
                             preferred_element_type=jnp.float32)
    h3 = jnp.maximum(h3 + b3_ref[...][None, :], 0.0)
    flag = flag_ref[0]
    h3m = h3 * flag + (flag - 1.0) * 1e30
    out_ref[0] = jnp.max(h3m.reshape(S_t, K, -1), axis=1)


def _mlp_pool(grouped, centroid_pad, flags3d, scale_params, K):
    B, S = grouped.shape[0], grouped.shape[1]
    (W1, b1), (W2, b2), (W3, b3) = scale_params
    W1p = jnp.pad(W1, ((0, 8 - W1.shape[0]), (0, 0)))
    h1, h2, h3 = W1.shape[1], W2.shape[1], W3.shape[1]
    S_t = max(2048 // K, 8)
    grid = (B, S // S_t)
    out = pl.pallas_call(
        functools.partial(_mlp_pool_body, S_t=S_t, K=K),
        grid=grid,
        in_specs=[
            pl.BlockSpec((1, S_t, K, 8), lambda b, s: (b, s, 0, 0)),
            pl.BlockSpec((1, S_t, 8), lambda b, s: (b, s, 0)),
            pl.BlockSpec((1, S_t * K, 1), lambda b, s: (b, s, 0)),
            pl.BlockSpec((8, h1), lambda b, s: (0, 0)),
            pl.BlockSpec((h1,), lambda b, s: (0,)),
            pl.BlockSpec((h1, h2), lambda b, s: (0, 0)),
            pl.BlockSpec((h2,), lambda b, s: (0,)),
            pl.BlockSpec((h2, h3), lambda b, s: (0, 0)),
            pl.BlockSpec((h3,), lambda b, s: (0,)),
        ],
        out_specs=pl.BlockSpec((1, S_t, h3), lambda b, s: (b, s, 0)),
        out_shape=jax.ShapeDtypeStruct((B, S, h3), jnp.float32),
    )(grouped, centroid_pad, flags3d, W1p, b1, W2, b2, W3, b3)
    return out


def kernel(x, params):
    B, N, C = x.shape
    xyz = x[:, :, :3]
    fps_idx = _fps(lax.stop_gradient(xyz), _N_OUT)
    bidx = jnp.arange(B)[:, None]
    centroid = xyz[bidx, fps_idx]
    centroid_pad = jnp.pad(centroid, ((0, 0), (0, 0), (0, 5)))
    xyzT = xyz.transpose(0, 2, 1)
    x8flat = jnp.pad(x, ((0, 0), (0, 0), (0, 2))).reshape(B * N * 8)
    g1, g2, g3, cnt1, cnt2, cnt3 = _ball_query_sc(
        lax.stop_gradient(centroid), lax.stop_gradient(xyzT), x8flat)
    results = []
    for g, cnt, n_sample, scale_params in zip(
            (g1, g2, g3), (cnt1, cnt2, cnt3), _N_SAMPLE, params):
        grouped = g.reshape(B, _N_OUT, n_sample, 8)
        flags3d = (jnp.arange(n_sample, dtype=jnp.int32)[None, None, :]
                   < jnp.maximum(cnt, 1)[:, :, None]).astype(jnp.float32)
        flags3d = flags3d.reshape(B, _N_OUT * n_sample, 1)
        results.append(
            _mlp_pool(grouped, centroid_pad, flags3d, scale_params, n_sample))
    return jnp.concatenate([centroid] + results, axis=2)


# speedup vs baseline: 3.7891x; 3.7891x over previous
import functools

import jax
import jax.numpy as jnp
import numpy as np
from jax import lax
from jax.experimental import pallas as pl
from jax.experimental.pallas import tpu as pltpu
from jax.experimental.pallas import tpu_sc as plsc

_N_OUT = 1024
_N_SAMPLE = [16, 32, 128]
_RADIUS = [0.1, 0.2, 0.4]

_NC, _NS, _L = 2, 16, 16
_NW = _NC * _NS


def _fps(xyz, K):
    B, N, _ = xyz.shape
    idx_out = jnp.zeros((B, K), dtype=jnp.int32)
    dists = jnp.full((B, N), 1e10, dtype=jnp.float32)
    farthest = jnp.zeros((B,), dtype=jnp.int32)

    def body(i, carry):
        idx_out, dists, farthest = carry
        idx_out = idx_out.at[:, i].set(farthest)
        cen = jnp.take_along_axis(xyz, farthest[:, None, None], axis=1)
        d = jnp.sum((xyz - cen) ** 2, axis=-1)
        dists = jnp.minimum(dists, d)
        farthest = jnp.argmax(dists, axis=-1).astype(jnp.int32)
        return (idx_out, dists, farthest)

    idx_out, _, _ = lax.fori_loop(0, K, body, (idx_out, dists, farthest))
    return idx_out


def _fps_body(xp_ref, yp_ref, zp_ref, cxo_ref, cyo_ref, czo_ref, *, K):
    xp = xp_ref[...]
    yp = yp_ref[...]
    zp = zp_ref[...]
    B, N = xp.shape
    iota_n = lax.broadcasted_iota(jnp.int32, (B, N), 1)
    eye = (lax.broadcasted_iota(jnp.int32, (B, B), 0)
           == lax.broadcasted_iota(jnp.int32, (B, B), 1))

    def body(i, carry):
        far, dists = carry
        onehot = iota_n == far
        cx = jnp.sum(jnp.where(onehot, xp, 0.0), axis=1, keepdims=True)
        cy = jnp.sum(jnp.where(onehot, yp, 0.0), axis=1, keepdims=True)
        cz = jnp.sum(jnp.where(onehot, zp, 0.0), axis=1, keepdims=True)
        cxo_ref[pl.ds(i, 1), :] = jnp.sum(
            jnp.where(eye, cx, 0.0), axis=0, keepdims=True)
        cyo_ref[pl.ds(i, 1), :] = jnp.sum(
            jnp.where(eye, cy, 0.0), axis=0, keepdims=True)
        czo_ref[pl.ds(i, 1), :] = jnp.sum(
            jnp.where(eye, cz, 0.0), axis=0, keepdims=True)
        dx = xp - cx
        dy = yp - cy
        dz = zp - cz
        d = dx * dx + dy * dy + dz * dz
        dists = jnp.minimum(dists, d)
        far = jnp.argmax(dists, axis=1, keepdims=True).astype(jnp.int32)
        return (far, dists)

    far0 = jnp.zeros((B, 1), jnp.int32)
    d0 = jnp.full((B, N), 1e10, jnp.float32)
    lax.fori_loop(0, K, body, (far0, d0))


def _fps_pallas(xyzT, K):
    B, _, N = xyzT.shape
    outs = pl.pallas_call(
        functools.partial(_fps_body, K=K),
        in_specs=[pl.BlockSpec((B, N), lambda: (0, 0))] * 3,
        out_specs=[pl.BlockSpec((K, B), lambda: (0, 0))] * 3,
        out_shape=[jax.ShapeDtypeStruct((K, B), jnp.float32)] * 3,
    )(xyzT[:, 0], xyzT[:, 1], xyzT[:, 2])
    return jnp.stack([o.T for o in outs], axis=-1)


_K1, _K2, _K3 = _N_SAMPLE
_R1SQ = np.float32(_RADIUS[0] * _RADIUS[0])
_R2SQ = np.float32(_RADIUS[1] * _RADIUS[1])
_R3SQ = np.float32(_RADIUS[2] * _RADIUS[2])


def _bq_body(cx_hbm, cy_hbm, cz_hbm, xp_hbm, yp_hbm, zp_hbm, x8_hbm,
             g1_hbm, g2_hbm, g3_hbm, cnt1_hbm, cnt2_hbm, cnt3_hbm,
             xv, yv, zv, x8v, cenx, ceny, cenz, o1, o2, o3,
             s1, s2, s3, c1s, c2s, c3s):
    B, N = xp_hbm.shape
    rows_per_w = (B * _N_OUT) // _NW
    groups = rows_per_w // _L
    wid = lax.axis_index("s") * _NC + lax.axis_index("c")
    b = wid // (_NW // B)
    row0 = wid * rows_per_w

    pltpu.sync_copy(xp_hbm.at[b], xv)
    pltpu.sync_copy(yp_hbm.at[b], yv)
    pltpu.sync_copy(zp_hbm.at[b], zv)
    pltpu.sync_copy(x8_hbm.at[pl.ds(b * N * 8, N * 8)], x8v)
    pltpu.sync_copy(cx_hbm.at[pl.ds(row0, rows_per_w)], cenx)
    pltpu.sync_copy(cy_hbm.at[pl.ds(row0, rows_per_w)], ceny)
    pltpu.sync_copy(cz_hbm.at[pl.ds(row0, rows_per_w)], cenz)

    zi = jnp.zeros((_L,), jnp.int32)
    for kk in range(_K1):
        o1[pl.ds(kk * _L, _L)] = zi
    for kk in range(_K2):
        o2[pl.ds(kk * _L, _L)] = zi
    for kk in range(_K3):
        o3[pl.ds(kk * _L, _L)] = zi

    lane = lax.iota(jnp.int32, _L)
    zeros16 = jnp.zeros((_L,), jnp.int32)

    def group_body(g, _):
        cxv = cenx[pl.ds(g * _L, _L)]
        cyv = ceny[pl.ds(g * _L, _L)]
        czv = cenz[pl.ds(g * _L, _L)]

        def point_body(n, carry):
            c1, c2, c3 = carry
            nv = jnp.full((_L,), n, dtype=jnp.int32)
            dx = cxv - plsc.load_gather(xv, [nv])
            dy = cyv - plsc.load_gather(yv, [nv])
            dz = czv - plsc.load_gather(zv, [nv])
            d2 = dx * dx + dy * dy + dz * dz
            sel1 = (d2 <= _R1SQ) & (c1 < _K1)
            sel2 = (d2 <= _R2SQ) & (c2 < _K2)
            sel3 = (d2 <= _R3SQ) & (c3 < _K3)
            plsc.store_scatter(
                o1, [lane * _K1 + jnp.minimum(c1, _K1 - 1)], nv, mask=sel1)
            plsc.store_scatter(
                o2, [lane * _K2 + jnp.minimum(c2, _K2 - 1)], nv, mask=sel2)
            plsc.store_scatter(
                o3, [lane * _K3 + jnp.minimum(c3, _K3 - 1)], nv, mask=sel3)
            return (c1 + sel1.astype(jnp.int32),
                    c2 + sel2.astype(jnp.int32),
                    c3 + sel3.astype(jnp.int32))

        z = jnp.zeros((_L,), jnp.int32)
        c1, c2, c3 = lax.fori_loop(0, N, point_body, (z, z, z))

        plsc.store_scatter(o1, [lane * _K1], zeros16, mask=(c1 == 0))
        plsc.store_scatter(o2, [lane * _K2], zeros16, mask=(c2 == 0))
        plsc.store_scatter(o3, [lane * _K3], zeros16, mask=(c3 == 0))

        def fill_row(r, _):
            for o, st, K in ((o1, s1, _K1), (o2, s2, _K2), (o3, s3, _K3)):
                for c in range(K // _L):
                    slots = o[pl.ds(r * K + c * _L, _L)]
                    a8 = slots * 8
                    wbase = (r * K + c * _L) * 8
                    for f in range(8):
                        vals = plsc.load_gather(x8v, [a8 + f])
                        plsc.store_scatter(st, [wbase + lane * 8 + f], vals)
            return 0

        lax.fori_loop(0, _L, fill_row, 0)

        c1s[pl.ds(0, _L)] = c1
        c2s[pl.ds(0, _L)] = c2
        c3s[pl.ds(0, _L)] = c3
        rbase = row0 + g * _L
        pltpu.sync_copy(s1, g1_hbm.at[pl.ds(rbase * _K1 * 8, _L * _K1 * 8)])
        pltpu.sync_copy(s2, g2_hbm.at[pl.ds(rbase * _K2 * 8, _L * _K2 * 8)])
        pltpu.sync_copy(s3, g3_hbm.at[pl.ds(rbase * _K3 * 8, _L * _K3 * 8)])
        pltpu.sync_copy(c1s, cnt1_hbm.at[pl.ds(rbase, _L)])
        pltpu.sync_copy(c2s, cnt2_hbm.at[pl.ds(rbase, _L)])
        pltpu.sync_copy(c3s, cnt3_hbm.at[pl.ds(rbase, _L)])
        return 0

    lax.fori_loop(0, groups, group_body, 0)


def _ball_query_sc(centroid, xyzT, x8flat):
    B, S, _ = centroid.shape
    N = xyzT.shape[2]
    BS = B * S
    cflat = centroid.reshape(BS, 3)
    mesh = plsc.VectorSubcoreMesh(core_axis_name="c", subcore_axis_name="s")
    out = pl.kernel(
        _bq_body,
        out_type=[
            jax.ShapeDtypeStruct((BS * _K1 * 8,), jnp.float32),
            jax.ShapeDtypeStruct((BS * _K2 * 8,), jnp.float32),
            jax.ShapeDtypeStruct((BS * _K3 * 8,), jnp.float32),
            jax.ShapeDtypeStruct((BS,), jnp.int32),
            jax.ShapeDtypeStruct((BS,), jnp.int32),
            jax.ShapeDtypeStruct((BS,), jnp.int32),
        ],
        mesh=mesh,
        compiler_params=pltpu.CompilerParams(needs_layout_passes=False),
        scratch_types=[
            pltpu.VMEM((N,), jnp.float32),
            pltpu.VMEM((N,), jnp.float32),
            pltpu.VMEM((N,), jnp.float32),
            pltpu.VMEM((N * 8,), jnp.float32),
            pltpu.VMEM((BS // _NW,), jnp.float32),
            pltpu.VMEM((BS // _NW,), jnp.float32),
            pltpu.VMEM((BS // _NW,), jnp.float32),
            pltpu.VMEM((_L * _K1,), jnp.int32),
            pltpu.VMEM((_L * _K2,), jnp.int32),
            pltpu.VMEM((_L * _K3,), jnp.int32),
            pltpu.VMEM((_L * _K1 * 8,), jnp.float32),
            pltpu.VMEM((_L * _K2 * 8,), jnp.float32),
            pltpu.VMEM((_L * _K3 * 8,), jnp.float32),
            pltpu.VMEM((_L,), jnp.int32),
            pltpu.VMEM((_L,), jnp.int32),
            pltpu.VMEM((_L,), jnp.int32),
        ],
    )(cflat[:, 0], cflat[:, 1], cflat[:, 2],
      xyzT[:, 0], xyzT[:, 1], xyzT[:, 2], x8flat)
    g1, g2, g3, cnt1, cnt2, cnt3 = out
    return (g1, g2, g3,
            cnt1.reshape(B, S), cnt2.reshape(B, S), cnt3.reshape(B, S))


def _mlp_pool_body(g_ref, c_ref, flag_ref, w1_ref, b1_ref, w2_ref, b2_ref,
                   w3_ref, b3_ref, out_ref, *, S_t, K):
    g = g_ref[0].reshape(S_t * K, 8)
    w1 = w1_ref[...]
    cc = jax.lax.dot_general(c_ref[0], w1, (((1,), (0,)), ((), ())),
                             preferred_element_type=jnp.float32)
    h = jax.lax.dot_general(g, w1, (((1,), (0,)), ((), ())),
                            preferred_element_type=jnp.float32)
    bias1 = b1_ref[...][None, :] - cc
    h1 = h.reshape(S_t, K, -1) + bias1[:, None, :]
    h1 = jnp.maximum(h1.reshape(S_t * K, -1), 0.0)
    h2 = jax.lax.dot_general(h1, w2_ref[...], (((1,), (0,)), ((), ())),
                             preferred_element_type=jnp.float32)
    h2 = jnp.maximum(h2 + b2_ref[...][None, :], 0.0)
    h3 = jax.lax.dot_general(h2, w3_ref[...], (((1,), (0,)), ((), ())),
                             preferred_element_type=jnp.float32)
    h3 = jnp.maximum(h3 + b3_ref[...][None, :], 0.0)
    flag = flag_ref[0]
    h3m = h3 * flag + (flag - 1.0) * 1e30
    out_ref[0] = jnp.max(h3m.reshape(S_t, K, -1), axis=1)


def _mlp_pool(grouped, centroid_pad, flags3d, scale_params, K):
    B, S = grouped.shape[0], grouped.shape[1]
    (W1, b1), (W2, b2), (W3, b3) = scale_params
    W1p = jnp.pad(W1, ((0, 8 - W1.shape[0]), (0, 0)))
    h1, h2, h3 = W1.shape[1], W2.shape[1], W3.shape[1]
    S_t = max(2048 // K, 8)
    grid = (B, S // S_t)
    out = pl.pallas_call(
        functools.partial(_mlp_pool_body, S_t=S_t, K=K),
        grid=grid,
        in_specs=[
            pl.BlockSpec((1, S_t, K, 8), lambda b, s: (b, s, 0, 0)),
            pl.BlockSpec((1, S_t, 8), lambda b, s: (b, s, 0)),
            pl.BlockSpec((1, S_t * K, 1), lambda b, s: (b, s, 0)),
            pl.BlockSpec((8, h1), lambda b, s: (0, 0)),
            pl.BlockSpec((h1,), lambda b, s: (0,)),
            pl.BlockSpec((h1, h2), lambda b, s: (0, 0)),
            pl.BlockSpec((h2,), lambda b, s: (0,)),
            pl.BlockSpec((h2, h3), lambda b, s: (0, 0)),
            pl.BlockSpec((h3,), lambda b, s: (0,)),
        ],
        out_specs=pl.BlockSpec((1, S_t, h3), lambda b, s: (b, s, 0)),
        out_shape=jax.ShapeDtypeStruct((B, S, h3), jnp.float32),
    )(grouped, centroid_pad, flags3d, W1p, b1, W2, b2, W3, b3)
    return out


def kernel(x, params):
    B, N, C = x.shape
    xyz = x[:, :, :3]
    xyzT = xyz.transpose(0, 2, 1)
    centroid = _fps_pallas(lax.stop_gradient(xyzT), _N_OUT)
    centroid_pad = jnp.pad(centroid, ((0, 0), (0, 0), (0, 5)))
    x8flat = jnp.pad(x, ((0, 0), (0, 0), (0, 2))).reshape(B * N * 8)
    g1, g2, g3, cnt1, cnt2, cnt3 = _ball_query_sc(
        lax.stop_gradient(centroid), lax.stop_gradient(xyzT), x8flat)
    results = []
    for g, cnt, n_sample, scale_params in zip(
            (g1, g2, g3), (cnt1, cnt2, cnt3), _N_SAMPLE, params):
        grouped = g.reshape(B, _N_OUT, n_sample, 8)
        flags3d = (jnp.arange(n_sample, dtype=jnp.int32)[None, None, :]
                   < jnp.maximum(cnt, 1)[:, :, None]).astype(jnp.float32)
        flags3d = flags3d.reshape(B, _N_OUT * n_sample, 1)
        results.append(
            _mlp_pool(grouped, centroid_pad, flags3d, scale_params, n_sample))
    return jnp.concatenate([centroid] + results, axis=2)
